# baseline (speedup 1.0000x reference)
import functools

import jax
import jax.numpy as jnp
from jax import lax
from jax.experimental import pallas as pl
from jax.experimental.pallas import tpu as pltpu
from jax.experimental.pallas import tpu_sc as plsc

NC = 2
NS = 16
L = 16
ROWS = 16
NBUF = 4
SC_BATCH = 1
S_BLK = 2048


def _sc_add(x, pe):
    batch, seq_len, d_model = x.shape
    nw = NC * NS
    seq_per_w = seq_len // nw
    n_steps = seq_per_w // ROWS
    nj = d_model // L
    mesh = plsc.VectorSubcoreMesh(core_axis_name="c", subcore_axis_name="s")

    @functools.partial(
        pl.kernel,
        mesh=mesh,
        out_type=jax.ShapeDtypeStruct((SC_BATCH, seq_len, d_model), x.dtype),
        scratch_types=[
            pltpu.VMEM((NBUF, ROWS, d_model), jnp.float32),
            pltpu.VMEM((NBUF, ROWS, d_model), jnp.float32),
            [pltpu.SemaphoreType.DMA] * NBUF,
            [pltpu.SemaphoreType.DMA] * NBUF,
            [pltpu.SemaphoreType.DMA] * NBUF,
        ],
    )
    def sc_add(x_hbm, pe_hbm, out_hbm, x_bufs, pe_bufs, sx, spe, so):
        wid = lax.axis_index("s") * NC + lax.axis_index("c")
        base = wid * seq_per_w

        def x_copy(i, p):
            rows = pl.ds(base + i * ROWS, ROWS)
            return pltpu.make_async_copy(x_hbm.at[0, rows, :], x_bufs.at[p], sx[p])

        def pe_copy(i, p):
            rows = pl.ds(base + i * ROWS, ROWS)
            return pltpu.make_async_copy(pe_hbm.at[rows, :], pe_bufs.at[p], spe[p])

        def out_copy(i, p):
            rows = pl.ds(base + i * ROWS, ROWS)
            return pltpu.make_async_copy(x_bufs.at[p], out_hbm.at[0, rows, :], so[p])

        def do_block(i, p):
            np_ = (p + 1) % NBUF
            @pl.when(i >= NBUF - 1)
            def _():
                out_copy(i - (NBUF - 1), np_).wait()

            @pl.when(i + 1 < n_steps)
            def _():
                x_copy(i + 1, np_).start()
                pe_copy(i + 1, np_).start()

            x_copy(i, p).wait()
            pe_copy(i, p).wait()

            def row_loop(r, c):
                for j in range(nj):
                    sl = pl.ds(j * L, L)
                    plsc.addupdate(x_bufs.at[p, r, sl], pe_bufs[p, r, sl])
                return c

            lax.fori_loop(0, ROWS, row_loop, 0)
            out_copy(i, p).start()

        def quad(k, c):
            for ph in range(NBUF):
                do_block(k * NBUF + ph, ph)
            return c

        x_copy(0, 0).start()
        pe_copy(0, 0).start()
        lax.fori_loop(0, n_steps // NBUF, quad, 0)
        for i in range(n_steps - (NBUF - 1), n_steps):
            out_copy(i, i % NBUF).wait()

    return sc_add(x, pe[:seq_len])


def _tc_body(x_ref, pe_ref, o_ref):
    o_ref[...] = x_ref[...] + pe_ref[...]


def _tc_add(x, pe):
    batch, seq_len, d_model = x.shape
    tc_batch = batch - SC_BATCH
    grid = (seq_len // S_BLK, tc_batch)
    return pl.pallas_call(
        _tc_body,
        grid=grid,
        in_specs=[
            pl.BlockSpec((1, S_BLK, d_model), lambda i, b: (b + SC_BATCH, i, 0)),
            pl.BlockSpec((S_BLK, d_model), lambda i, b: (i, 0)),
        ],
        out_specs=pl.BlockSpec((1, S_BLK, d_model), lambda i, b: (b, i, 0)),
        out_shape=jax.ShapeDtypeStruct((tc_batch, seq_len, d_model), x.dtype),
        compiler_params=pltpu.CompilerParams(
            dimension_semantics=("arbitrary", "arbitrary"),
        ),
    )(x, pe[:seq_len])


def kernel(x, pe):
    out_sc = _sc_add(x, pe)
    out_tc = _tc_add(x, pe)
    return jnp.concatenate([out_sc, out_tc], axis=0)
